# Initial kernel scaffold; baseline (speedup 1.0000x reference)
import jax
import jax.numpy as jnp
from jax import lax
from jax.experimental import pallas as pl
from jax.experimental.pallas import tpu as pltpu
from jax.experimental.pallas import tpu_sc as plsc

BATCH = 16384
N_FIELDS = 26
DIM = 32
B_TOTAL = BATCH * N_FIELDS

NC = 2
NS = 16
NW = NC * NS
B_PER_W = B_TOTAL // NW
CHUNK = 1024
N_CHUNKS = B_PER_W // CHUNK


def _gather_body(table_hbm, idx_hbm, out_hbm, idx_v, rows_v, sem):
    wid = lax.axis_index("s") * NC + lax.axis_index("c")
    base = wid * B_PER_W
    for i in range(N_CHUNKS):
        row0 = base + i * CHUNK
        pltpu.sync_copy(idx_hbm.at[pl.ds(row0, CHUNK)], idx_v)
        pltpu.async_copy(table_hbm.at[idx_v], rows_v, sem).wait()
        pltpu.sync_copy(rows_v, out_hbm.at[pl.ds(row0, CHUNK)])


@jax.jit
def _lookup(idx, embedding):
    mesh = plsc.VectorSubcoreMesh(core_axis_name="c", subcore_axis_name="s")
    run = pl.kernel(
        _gather_body,
        mesh=mesh,
        out_type=jax.ShapeDtypeStruct((B_TOTAL, DIM), jnp.float32),
        scratch_types=[
            pltpu.VMEM((CHUNK,), jnp.int32),
            pltpu.VMEM((CHUNK, DIM), jnp.float32),
            pltpu.SemaphoreType.DMA,
        ],
    )
    return run(embedding, idx)


def kernel(inputs, embedding):
    idx = jnp.reshape(inputs, (B_TOTAL,)).astype(jnp.int32)
    out = _lookup(idx, embedding)
    return out.reshape(BATCH, N_FIELDS, DIM)


# speedup vs baseline: 1.5471x; 1.5471x over previous
import jax
import jax.numpy as jnp
from jax import lax
from jax.experimental import pallas as pl
from jax.experimental.pallas import tpu as pltpu
from jax.experimental.pallas import tpu_sc as plsc

BATCH = 16384
N_FIELDS = 26
DIM = 32
B_TOTAL = BATCH * N_FIELDS

NC = 2
NS = 16
NW = NC * NS
B_PER_W = B_TOTAL // NW
CHUNK = 1024
N_CHUNKS = B_PER_W // CHUNK


def _gather_body(table_hbm, idx_hbm, out_hbm, idx_v, rows_v, sem):
    wid = lax.axis_index("s") * NC + lax.axis_index("c")
    base = wid * B_PER_W
    for i in range(N_CHUNKS):
        row0 = base + i * CHUNK
        pltpu.sync_copy(idx_hbm.at[pl.ds(row0, CHUNK)], idx_v)
        pltpu.async_copy(table_hbm.at[idx_v], rows_v, sem).wait()
        pltpu.sync_copy(rows_v, out_hbm.at[pl.ds(row0, CHUNK)])


@jax.jit
def _lookup(idx, embedding):
    mesh = plsc.VectorSubcoreMesh(core_axis_name="c", subcore_axis_name="s")
    run = pl.kernel(
        _gather_body,
        mesh=mesh,
        out_type=jax.ShapeDtypeStruct((B_TOTAL, DIM), jnp.float32),
        scratch_types=[
            pltpu.VMEM((CHUNK,), jnp.int32),
            pltpu.VMEM((CHUNK, DIM), jnp.float32),
            pltpu.SemaphoreType.DMA,
        ],
        compiler_params=pltpu.CompilerParams(use_tc_tiling_on_sc=False),
    )
    return run(embedding, idx)


def kernel(inputs, embedding):
    idx = jnp.reshape(inputs, (B_TOTAL,)).astype(jnp.int32)
    out = _lookup(idx, embedding)
    return out.reshape(BATCH, N_FIELDS, DIM)


# speedup vs baseline: 1.5794x; 1.0209x over previous
import jax
import jax.numpy as jnp
from jax import lax
from jax.experimental import pallas as pl
from jax.experimental.pallas import tpu as pltpu
from jax.experimental.pallas import tpu_sc as plsc

BATCH = 16384
N_FIELDS = 26
DIM = 32
B_TOTAL = BATCH * N_FIELDS

NC = 2
NS = 16
NW = NC * NS
B_PER_W = B_TOTAL // NW
CHUNK = 1024
N_CHUNKS = B_PER_W // CHUNK
NBUF = 3


def _gather_body(table_hbm, idx_hbm, out_hbm, idx_v, rows0, rows1, rows2,
                 g0, g1, g2, s0, s1, s2):
    rows = (rows0, rows1, rows2)
    gsem = (g0, g1, g2)
    ssem = (s0, s1, s2)
    wid = lax.axis_index("s") * NC + lax.axis_index("c")
    base = wid * B_PER_W

    pltpu.sync_copy(idx_hbm.at[pl.ds(base, B_PER_W)], idx_v)

    def idx_slice(i):
        return idx_v.at[pl.ds(i * CHUNK, CHUNK)]

    for i in range(min(NBUF, N_CHUNKS)):
        pltpu.async_copy(table_hbm.at[idx_slice(i)], rows[i], gsem[i])
    for i in range(N_CHUNKS):
        b = i % NBUF
        pltpu.make_async_copy(table_hbm.at[idx_slice(i)], rows[b], gsem[b]).wait()
        out_sl = out_hbm.at[pl.ds(base + i * CHUNK, CHUNK)]
        pltpu.async_copy(rows[b], out_sl, ssem[b])
        j = i + NBUF
        if j < N_CHUNKS:
            pltpu.make_async_copy(rows[b], out_sl, ssem[b]).wait()
            pltpu.async_copy(table_hbm.at[idx_slice(j)], rows[b], gsem[b])
    for i in range(max(0, N_CHUNKS - NBUF), N_CHUNKS):
        b = i % NBUF
        out_sl = out_hbm.at[pl.ds(base + i * CHUNK, CHUNK)]
        pltpu.make_async_copy(rows[b], out_sl, ssem[b]).wait()


@jax.jit
def _lookup(idx, embedding):
    mesh = plsc.VectorSubcoreMesh(core_axis_name="c", subcore_axis_name="s")
    run = pl.kernel(
        _gather_body,
        mesh=mesh,
        out_type=jax.ShapeDtypeStruct((B_TOTAL, DIM), jnp.float32),
        scratch_types=[
            pltpu.VMEM((B_PER_W,), jnp.int32),
            pltpu.VMEM((CHUNK, DIM), jnp.float32),
            pltpu.VMEM((CHUNK, DIM), jnp.float32),
            pltpu.VMEM((CHUNK, DIM), jnp.float32),
            pltpu.SemaphoreType.DMA,
            pltpu.SemaphoreType.DMA,
            pltpu.SemaphoreType.DMA,
            pltpu.SemaphoreType.DMA,
            pltpu.SemaphoreType.DMA,
            pltpu.SemaphoreType.DMA,
        ],
        compiler_params=pltpu.CompilerParams(use_tc_tiling_on_sc=False),
    )
    return run(embedding, idx)


def kernel(inputs, embedding):
    idx = jnp.reshape(inputs, (B_TOTAL,)).astype(jnp.int32)
    out = _lookup(idx, embedding)
    return out.reshape(BATCH, N_FIELDS, DIM)
